# baseline (speedup 1.0000x reference)
import jax
import jax.numpy as jnp
from jax import lax
from jax.experimental import pallas as pl
from jax.experimental.pallas import tpu as pltpu
from jax.experimental.pallas import tpu_sc as plsc

B = 16384
D = 64
V = 1000000
NC = 2
NS = 16
NW = NC * NS
BPW = B // NW
C = 128

RB = 512
PB = 4000


def _tc_widen_body(t_ref, out_ref):
    out_ref[:, 0:D] = t_ref[...]


def _tc_widen(t):
    return pl.pallas_call(
        _tc_widen_body,
        grid=(V // PB,),
        in_specs=[pl.BlockSpec((PB, D), lambda i: (i, 0))],
        out_specs=pl.BlockSpec((PB, 2 * D), lambda i: (i, 0)),
        out_shape=jax.ShapeDtypeStruct((V, 2 * D), jnp.float32),
    )(t)


def _sc_gather_body(i4h, i5h, t1, t2, ru2_out, ri2_out,
                    i4_v, i5_v, r4_v, r5_v, sem4, sem5):
    wid = lax.axis_index("s") * NC + lax.axis_index("c")
    base = wid * BPW

    pltpu.sync_copy(i5h.at[pl.ds(base, BPW)], i5_v)
    pltpu.sync_copy(i4h.at[pl.ds(base, BPW)], i4_v)

    @pl.loop(0, BPW, step=C)
    def _(c):
        g5 = pltpu.async_copy(t2.at[i5_v.at[pl.ds(c, C)]], r5_v, sem5)
        g4 = pltpu.async_copy(t1.at[i4_v.at[pl.ds(c, C)]], r4_v, sem4)
        g5.wait()
        pltpu.sync_copy(r5_v, ri2_out.at[pl.ds(base + c, C)])
        g4.wait()
        pltpu.sync_copy(r4_v, ru2_out.at[pl.ds(base + c, C)])


def _tc_concat_body(ri2_ref, cv0_ref, ru2_ref, cv3_ref, out_ref):
    out_ref[:, 0 * D:1 * D] = ri2_ref[:, 0:D]
    out_ref[:, 1 * D:2 * D] = cv0_ref[...]
    out_ref[:, 2 * D:3 * D] = ru2_ref[:, 0:D]
    out_ref[:, 3 * D:4 * D] = cv3_ref[...]


def kernel(adj_list_4, adj_list_5, concat_vecs_0, concat_vecs_1,
           concat_vecs_2, concat_vecs_3):
    i4 = adj_list_4.astype(jnp.int32)
    i5 = adj_list_5.astype(jnp.int32)
    t1 = _tc_widen(concat_vecs_1)
    t2 = _tc_widen(concat_vecs_2)

    mesh = plsc.VectorSubcoreMesh(core_axis_name="c", subcore_axis_name="s")
    gather_k = pl.kernel(
        _sc_gather_body,
        out_type=(jax.ShapeDtypeStruct((B, 2 * D), jnp.float32),
                  jax.ShapeDtypeStruct((B, 2 * D), jnp.float32)),
        mesh=mesh,
        scratch_types=[
            pltpu.VMEM((BPW,), jnp.int32),
            pltpu.VMEM((BPW,), jnp.int32),
            pltpu.VMEM((C, 2 * D), jnp.float32),
            pltpu.VMEM((C, 2 * D), jnp.float32),
            pltpu.SemaphoreType.DMA,
            pltpu.SemaphoreType.DMA,
        ],
    )
    ru2, ri2 = gather_k(i4, i5, t1, t2)

    spec64 = pl.BlockSpec((RB, D), lambda i: (i, 0))
    spec128 = pl.BlockSpec((RB, 2 * D), lambda i: (i, 0))
    out = pl.pallas_call(
        _tc_concat_body,
        grid=(B // RB,),
        in_specs=[spec128, spec64, spec128, spec64],
        out_specs=pl.BlockSpec((RB, 4 * D), lambda i: (i, 0)),
        out_shape=jax.ShapeDtypeStruct((B, 4 * D), jnp.float32),
    )(ri2, concat_vecs_0, ru2, concat_vecs_3)
    return out


# speedup vs baseline: 1.3234x; 1.3234x over previous
import jax
import jax.numpy as jnp
from jax import lax
from jax.experimental import pallas as pl
from jax.experimental.pallas import tpu as pltpu
from jax.experimental.pallas import tpu_sc as plsc

B = 16384
D = 64
V = 1000000
NC = 2
NS = 16
NW = NC * NS
BPW = B // NW
C = 128

RB = 512


def _sc_gather_body(i4h, i5h, t1, t2, ru2_out, ri2_out,
                    i4_v, i5_v, r4_v, r5_v, sem4, sem5):
    wid = lax.axis_index("s") * NC + lax.axis_index("c")
    base = wid * BPW

    pltpu.sync_copy(i5h.at[pl.ds(base, BPW)], i5_v)
    pltpu.sync_copy(i4h.at[pl.ds(base, BPW)], i4_v)

    @pl.loop(0, BPW, step=C)
    def _(c):
        g5 = pltpu.async_copy(t2.at[i5_v.at[pl.ds(c, C)]], r5_v, sem5)
        g4 = pltpu.async_copy(t1.at[i4_v.at[pl.ds(c, C)]], r4_v, sem4)
        g5.wait()
        pltpu.sync_copy(r5_v, ri2_out.at[pl.ds(base + c, C)])
        g4.wait()
        pltpu.sync_copy(r4_v, ru2_out.at[pl.ds(base + c, C)])


def _tc_concat_body(ri2_ref, cv0_ref, ru2_ref, cv3_ref, out_ref):
    out_ref[:, 0 * D:1 * D] = ri2_ref[:, 0:D]
    out_ref[:, 1 * D:2 * D] = cv0_ref[...]
    out_ref[:, 2 * D:3 * D] = ru2_ref[:, 0:D]
    out_ref[:, 3 * D:4 * D] = cv3_ref[...]


def kernel(adj_list_4, adj_list_5, concat_vecs_0, concat_vecs_1,
           concat_vecs_2, concat_vecs_3):
    i4 = adj_list_4.astype(jnp.int32)
    i5 = adj_list_5.astype(jnp.int32)
    t1 = jnp.pad(concat_vecs_1, ((0, 0), (0, D)))
    t2 = jnp.pad(concat_vecs_2, ((0, 0), (0, D)))

    mesh = plsc.VectorSubcoreMesh(core_axis_name="c", subcore_axis_name="s")
    gather_k = pl.kernel(
        _sc_gather_body,
        out_type=(jax.ShapeDtypeStruct((B, 2 * D), jnp.float32),
                  jax.ShapeDtypeStruct((B, 2 * D), jnp.float32)),
        mesh=mesh,
        scratch_types=[
            pltpu.VMEM((BPW,), jnp.int32),
            pltpu.VMEM((BPW,), jnp.int32),
            pltpu.VMEM((C, 2 * D), jnp.float32),
            pltpu.VMEM((C, 2 * D), jnp.float32),
            pltpu.SemaphoreType.DMA,
            pltpu.SemaphoreType.DMA,
        ],
    )
    ru2, ri2 = gather_k(i4, i5, t1, t2)

    spec64 = pl.BlockSpec((RB, D), lambda i: (i, 0))
    spec128 = pl.BlockSpec((RB, 2 * D), lambda i: (i, 0))
    out = pl.pallas_call(
        _tc_concat_body,
        grid=(B // RB,),
        in_specs=[spec128, spec64, spec128, spec64],
        out_specs=pl.BlockSpec((RB, 4 * D), lambda i: (i, 0)),
        out_shape=jax.ShapeDtypeStruct((B, 4 * D), jnp.float32),
    )(ri2, concat_vecs_0, ru2, concat_vecs_3)
    return out
